# Initial kernel scaffold; baseline (speedup 1.0000x reference)
import jax
import jax.numpy as jnp
from jax.experimental import pallas as pl


def kernel(item_indices, item_embeddings):
    raise NotImplementedError("write your pallas kernel here")


# speedup vs baseline: 1.7976x; 1.7976x over previous
import functools

import jax
import jax.numpy as jnp
from jax import lax
from jax.experimental import pallas as pl
from jax.experimental.pallas import tpu as pltpu
from jax.experimental.pallas import tpu_sc as plsc

_BATCH = 16384
_SEQ = 50
_DIM = 64
_NUM_IDX = _BATCH * _SEQ
_NC = 2
_NS = 16
_NW = _NC * _NS
_PER_W = _NUM_IDX // _NW
_CHUNK = 512
_STEPS = _PER_W // _CHUNK


def _gather_sc(table, idx_flat):
    mesh = plsc.VectorSubcoreMesh(core_axis_name="c", subcore_axis_name="s")

    @functools.partial(
        pl.kernel,
        mesh=mesh,
        out_type=jax.ShapeDtypeStruct((_NUM_IDX, _DIM), table.dtype),
        scratch_types=[
            pltpu.VMEM((_CHUNK,), jnp.int32),
            pltpu.VMEM((_CHUNK, _DIM), table.dtype),
            pltpu.SemaphoreType.DMA,
        ],
        compiler_params=pltpu.CompilerParams(use_tc_tiling_on_sc=False),
    )
    def gather_kernel(table_hbm, idx_hbm, out_hbm, idx_v, rows_v, sem):
        wid = lax.axis_index("s") * _NC + lax.axis_index("c")
        base = wid * _PER_W

        @pl.loop(0, _STEPS)
        def _(c):
            off = base + c * _CHUNK
            pltpu.sync_copy(idx_hbm.at[pl.ds(off, _CHUNK)], idx_v)
            pltpu.async_copy(table_hbm.at[idx_v], rows_v, sem).wait()
            pltpu.sync_copy(rows_v, out_hbm.at[pl.ds(off, _CHUNK)])

    return gather_kernel(table, idx_flat)


def kernel(item_indices, item_embeddings):
    idx_flat = item_indices.reshape(_NUM_IDX).astype(jnp.int32)
    out = _gather_sc(item_embeddings, idx_flat)
    return out.reshape(_BATCH, _SEQ, _DIM)


# speedup vs baseline: 1.8750x; 1.0431x over previous
import functools

import jax
import jax.numpy as jnp
from jax import lax
from jax.experimental import pallas as pl
from jax.experimental.pallas import tpu as pltpu
from jax.experimental.pallas import tpu_sc as plsc

_BATCH = 16384
_SEQ = 50
_DIM = 64
_NUM_IDX = _BATCH * _SEQ
_NC = 2
_NS = 16
_NW = _NC * _NS
_PER_W = _NUM_IDX // _NW
_CHUNK = 512
_STEPS = _PER_W // _CHUNK
_NBUF = 2


def _gather_sc(table, idx_flat):
    mesh = plsc.VectorSubcoreMesh(core_axis_name="c", subcore_axis_name="s")

    @functools.partial(
        pl.kernel,
        mesh=mesh,
        out_type=jax.ShapeDtypeStruct((_NUM_IDX, _DIM), table.dtype),
        scratch_types=[
            pltpu.VMEM((_PER_W,), jnp.int32),
            [pltpu.VMEM((_CHUNK, _DIM), table.dtype) for _ in range(_NBUF)],
            [pltpu.SemaphoreType.DMA for _ in range(_NBUF)],
            [pltpu.SemaphoreType.DMA for _ in range(_NBUF)],
        ],
        compiler_params=pltpu.CompilerParams(use_tc_tiling_on_sc=False),
    )
    def gather_kernel(table_hbm, idx_hbm, out_hbm, idx_all, rows, gsem, wsem):
        wid = lax.axis_index("s") * _NC + lax.axis_index("c")
        base = wid * _PER_W
        pltpu.sync_copy(idx_hbm.at[pl.ds(base, _PER_W)], idx_all)

        def start_gather(s, b):
            pltpu.async_copy(
                table_hbm.at[idx_all.at[pl.ds(s * _CHUNK, _CHUNK)]],
                rows[b], gsem[b])

        def wait_gather(s, b):
            pltpu.make_async_copy(
                table_hbm.at[idx_all.at[pl.ds(s * _CHUNK, _CHUNK)]],
                rows[b], gsem[b]).wait()

        def start_write(s, b):
            pltpu.async_copy(rows[b], out_hbm.at[pl.ds(base + s * _CHUNK,
                                                       _CHUNK)], wsem[b])

        def wait_write(s, b):
            pltpu.make_async_copy(rows[b],
                                  out_hbm.at[pl.ds(base + s * _CHUNK, _CHUNK)],
                                  wsem[b]).wait()

        for b in range(_NBUF):
            start_gather(b, b)

        @pl.loop(0, _STEPS - _NBUF, step=_NBUF)
        def _(c):
            for b in range(_NBUF):
                s = c + b
                wait_gather(s, b)
                start_write(s, b)
                wait_write(s, b)
                start_gather(s + _NBUF, b)

        for b in range(_NBUF):
            s = _STEPS - _NBUF + b
            wait_gather(s, b)
            start_write(s, b)
            wait_write(s, b)

    return gather_kernel(table, idx_flat)


def kernel(item_indices, item_embeddings):
    idx_flat = item_indices.reshape(_NUM_IDX).astype(jnp.int32)
    out = _gather_sc(item_embeddings, idx_flat)
    return out.reshape(_BATCH, _SEQ, _DIM)
